# Initial kernel scaffold; baseline (speedup 1.0000x reference)
import jax
import jax.numpy as jnp
from jax.experimental import pallas as pl


def kernel(atom_xyz, probe_xyz, cell, probe_edges, probe_edges_displacement, num_nodes, num_probes, num_probe_edges, S_JK, V_JK, Ws1, bs1, ln_g, ln_b, Ws2, bs2, Wf1, bf1, Wf2, bf2, Wo1, bo1, Wo2, bo2, final_bias):
    raise NotImplementedError("write your pallas kernel here")


# speedup vs baseline: 5.1824x; 5.1824x over previous
import functools
import math

import jax
import jax.numpy as jnp
from jax import lax
from jax.experimental import pallas as pl
from jax.experimental.pallas import tpu as pltpu
from jax.experimental.pallas import tpu_sc as plsc

B = 4
N_MAX = 200
P_MAX = 4000
E_MAX = 50000
F = 64
N_SINC = 20
CUTOFF = 4.0

NPA = 256
ET = 1024
TPB = 49
EPB = TPB * ET
EP = B * EPB
GRID = B * TPB
NW = 32
EPW = EP // NW
CH = 128
NCH = EPW // CH
PT = B * P_MAX
AT = B * NPA
CW = 16

_f32 = jnp.float32
_i32 = jnp.int32


def _sc_mesh():
    return plsc.VectorSubcoreMesh(core_axis_name="c", subcore_axis_name="s")


def _gather_coords(ptab, atab, pidx, aidx):
    @functools.partial(
        pl.kernel,
        out_type=(
            jax.ShapeDtypeStruct((EP, CW), _f32),
            jax.ShapeDtypeStruct((EP, CW), _f32),
        ),
        mesh=_sc_mesh(),
        scratch_types=[
            pltpu.VMEM((NCH, CH), _i32),
            pltpu.VMEM((NCH, CH), _i32),
            pltpu.VMEM((CH, CW), _f32),
            pltpu.VMEM((CH, CW), _f32),
            pltpu.SemaphoreType.DMA,
            pltpu.SemaphoreType.DMA,
        ],
        compiler_params=pltpu.CompilerParams(use_tc_tiling_on_sc=False),
    )
    def k(ptab_h, atab_h, pidx_h, aidx_h, pe_h, ae_h, pidx_v, aidx_v, pbuf, abuf, psem, asem):
        c = lax.axis_index("c")
        s = lax.axis_index("s")
        wid = s * 2 + c
        pltpu.sync_copy(pidx_h.at[wid], pidx_v)
        pltpu.sync_copy(aidx_h.at[wid], aidx_v)
        base = wid * EPW

        def body(j, carry):
            cp = pltpu.async_copy(ptab_h.at[pidx_v.at[j]], pbuf, psem)
            ca = pltpu.async_copy(atab_h.at[aidx_v.at[j]], abuf, asem)
            cp.wait()
            ca.wait()
            pltpu.sync_copy(pbuf, pe_h.at[pl.ds(base + j * CH, CH)])
            pltpu.sync_copy(abuf, ae_h.at[pl.ds(base + j * CH, CH)])
            return carry

        lax.fori_loop(0, NCH, body, 0)

    return k(ptab, atab, pidx, aidx)


def _scatter_rho(pidx, vals, zeros):
    @functools.partial(
        pl.kernel,
        out_type=jax.ShapeDtypeStruct((2, PT), _f32),
        mesh=_sc_mesh(),
        scratch_types=[
            pltpu.VMEM((NCH, CH), _i32),
            pltpu.VMEM((NCH, CH), _f32),
            pltpu.VMEM_SHARED((PT,), _f32),
        ],
    )
    def k(pidx_h, vals_h, zeros_h, out_h, pidx_v, vals_v, rho_sh):
        c = lax.axis_index("c")
        s = lax.axis_index("s")
        wid = s * 2 + c

        @pl.when(s == 0)
        def _init():
            pltpu.sync_copy(zeros_h, rho_sh)

        plsc.subcore_barrier()
        pltpu.sync_copy(pidx_h.at[wid], pidx_v)
        pltpu.sync_copy(vals_h.at[wid], vals_v)

        def body(j, carry):
            pltpu.sync_copy(vals_v.at[j], rho_sh.at[pidx_v.at[j]], add=True)
            return carry

        lax.fori_loop(0, NCH, body, 0)
        plsc.subcore_barrier()

        @pl.when(s == 0)
        def _emit():
            pltpu.sync_copy(rho_sh, out_h.at[c])

    return k(pidx, vals, zeros)


def _tc_dense(aidx3, pe3, ae3, S_p, V_p, Ws1, bs1, ln_g, ln_b, Ws2, bs2,
              Wf1, bf1, Wf2, bf2, Wo1, bo1, Wo2, bo2):
    def body(aidx_ref, pe_ref, ae_ref, S_ref, V_ref, Ws1_ref, bs1_ref,
             lng_ref, lnb_ref, Ws2_ref, bs2_ref, Wf1_ref, bf1_ref, Wf2_ref,
             bf2_ref, Wo1_ref, bo1_ref, Wo2_ref, bo2_ref, mw_ref):
        i = pl.program_id(0)
        t = i % TPB

        V0 = V_ref[0, 0]
        V1 = V_ref[0, 1]
        V2 = V_ref[0, 2]
        S = S_ref[0]
        n_at = jnp.sqrt(V0 * V0 + V1 * V1 + V2 * V2 + 1e-8)
        WA = Ws1_ref[0:64, :]
        WB = Ws1_ref[64:128, :]
        WC = Ws1_ref[128:192, :]
        A1 = (jnp.dot(S, WA, preferred_element_type=_f32)
              + jnp.dot(n_at, WB, preferred_element_type=_f32)
              + bs1_ref[0:1, :])
        VW0 = jnp.dot(V0, WC, preferred_element_type=_f32)
        VW1 = jnp.dot(V1, WC, preferred_element_type=_f32)
        VW2 = jnp.dot(V2, WC, preferred_element_type=_f32)

        a = aidx_ref[0, 0, :]
        onehot = (a[:, None] == lax.broadcasted_iota(_i32, (1, NPA), 1)
                  ).astype(_f32)
        Ga = jnp.dot(onehot, A1, preferred_element_type=_f32)
        G0 = jnp.dot(onehot, VW0, preferred_element_type=_f32)
        G1 = jnp.dot(onehot, VW1, preferred_element_type=_f32)
        G2 = jnp.dot(onehot, VW2, preferred_element_type=_f32)

        diff = pe_ref[0] - ae_ref[0]
        d2 = jnp.sum(diff * diff, axis=1, keepdims=True)
        dist = jnp.sqrt(d2)
        inv_safe = 1.0 / jnp.sqrt(d2 + 1e-8)
        r0 = diff[:, 0:1] * inv_safe
        r1 = diff[:, 1:2] * inv_safe
        r2 = diff[:, 2:3] * inv_safe

        u = Ga + r0 * G0 + r1 * G1 + r2 * G2
        mu = jnp.mean(u, axis=1, keepdims=True)
        var = jnp.mean((u - mu) ** 2, axis=1, keepdims=True)
        y = (u - mu) / jnp.sqrt(var + 1e-5) * lng_ref[0:1, :] + lnb_ref[0:1, :]
        sy = y * (1.0 / (1.0 + jnp.exp(-y)))
        state = jnp.dot(sy, Ws2_ref[...], preferred_element_type=_f32) + bs2_ref[0:1, :]

        kfreq = ((lax.broadcasted_iota(_i32, (1, N_SINC), 1) + 1).astype(_f32)
                 * (math.pi / CUTOFF))
        e_pi = jnp.sin(dist * kfreq) / dist
        f1 = jnp.dot(e_pi, Wf1_ref[...], preferred_element_type=_f32) + bf1_ref[0:1, :]
        sf1 = f1 * (1.0 / (1.0 + jnp.exp(-f1)))
        W_pi = jnp.dot(sf1, Wf2_ref[...], preferred_element_type=_f32) + bf2_ref[0:1, :]

        m_prime = W_pi * state
        o1 = jnp.dot(m_prime, Wo1_ref[...], preferred_element_type=_f32) + bo1_ref[0:1, :]
        so1 = o1 * (1.0 / (1.0 + jnp.exp(-o1)))
        m = jnp.dot(so1, Wo2_ref[...], preferred_element_type=_f32) + bo2_ref[0:1, :]

        x = dist * (1.0 / CUTOFF)
        x5 = x * x * x * x * x
        env = 1.0 - 21.0 * x5 + 35.0 * x5 * x - 15.0 * x5 * x * x
        env = jnp.where(dist < CUTOFF, env, 0.0)

        mw = m * env
        eid = t * ET + lax.broadcasted_iota(_i32, (ET, 1), 0)
        mw = jnp.where(eid < E_MAX, mw, 0.0)
        mw_ref[0, 0, :] = mw[:, 0]

    grid_spec = pl.GridSpec(
        grid=(GRID,),
        in_specs=[
            pl.BlockSpec((1, 1, ET), lambda i: (i, 0, 0)),
            pl.BlockSpec((1, ET, CW), lambda i: (i, 0, 0)),
            pl.BlockSpec((1, ET, CW), lambda i: (i, 0, 0)),
            pl.BlockSpec((1, NPA, F), lambda i: (i // TPB, 0, 0)),
            pl.BlockSpec((1, 3, NPA, F), lambda i: (i // TPB, 0, 0, 0)),
            pl.BlockSpec((192, F), lambda i: (0, 0)),
            pl.BlockSpec((1, F), lambda i: (0, 0)),
            pl.BlockSpec((1, F), lambda i: (0, 0)),
            pl.BlockSpec((1, F), lambda i: (0, 0)),
            pl.BlockSpec((F, F), lambda i: (0, 0)),
            pl.BlockSpec((1, F), lambda i: (0, 0)),
            pl.BlockSpec((N_SINC, F), lambda i: (0, 0)),
            pl.BlockSpec((1, F), lambda i: (0, 0)),
            pl.BlockSpec((F, F), lambda i: (0, 0)),
            pl.BlockSpec((1, F), lambda i: (0, 0)),
            pl.BlockSpec((F, F // 2), lambda i: (0, 0)),
            pl.BlockSpec((1, F // 2), lambda i: (0, 0)),
            pl.BlockSpec((F // 2, 1), lambda i: (0, 0)),
            pl.BlockSpec((1, 1), lambda i: (0, 0)),
        ],
        out_specs=pl.BlockSpec((1, 1, ET), lambda i: (i, 0, 0)),
    )

    return pl.pallas_call(
        body,
        grid_spec=grid_spec,
        out_shape=jax.ShapeDtypeStruct((GRID, 1, ET), _f32),
    )(aidx3, pe3, ae3, S_p, V_p, Ws1, bs1, ln_g, ln_b, Ws2, bs2,
      Wf1, bf1, Wf2, bf2, Wo1, bo1, Wo2, bo2)


def kernel(atom_xyz, probe_xyz, cell, probe_edges, probe_edges_displacement,
           num_nodes, num_probes, num_probe_edges, S_JK, V_JK, Ws1, bs1,
           ln_g, ln_b, Ws2, bs2, Wf1, bf1, Wf2, bf2, Wo1, bo1, Wo2, bo2,
           final_bias):
    pad_e = EPB - E_MAX

    a_idx = probe_edges[:, :, 0].astype(_i32)
    p_idx = probe_edges[:, :, 1].astype(_i32)
    a_idx = jnp.pad(a_idx, ((0, 0), (0, pad_e)))
    p_idx = jnp.pad(p_idx, ((0, 0), (0, pad_e)))
    boff = jnp.arange(B, dtype=_i32)[:, None]
    aidx_g = (a_idx + boff * NPA).reshape(EP)
    pidx_g = (p_idx + boff * P_MAX).reshape(EP)

    ptab = jnp.pad(probe_xyz.reshape(PT, 3), ((0, 0), (0, CW - 3)))
    atab = jnp.pad(atom_xyz, ((0, 0), (0, NPA - N_MAX), (0, CW - 3))).reshape(AT, CW)

    pidx_w = pidx_g.reshape(NW, NCH, CH)
    aidx_w = aidx_g.reshape(NW, NCH, CH)
    pe, ae = _gather_coords(ptab, atab, pidx_w, aidx_w)

    S_p = jnp.pad(S_JK.reshape(B, N_MAX, F), ((0, 0), (0, NPA - N_MAX), (0, 0)))
    V_p = jnp.pad(V_JK.reshape(B, N_MAX, 3, F).transpose(0, 2, 1, 3),
                  ((0, 0), (0, 0), (0, NPA - N_MAX), (0, 0)))

    aidx3 = a_idx.reshape(GRID, 1, ET)
    pe3 = pe.reshape(GRID, ET, CW)
    ae3 = ae.reshape(GRID, ET, CW)

    mw = _tc_dense(
        aidx3, pe3, ae3, S_p, V_p, Ws1,
        bs1.reshape(1, F), ln_g.reshape(1, F), ln_b.reshape(1, F),
        Ws2, bs2.reshape(1, F), Wf1, bf1.reshape(1, F), Wf2,
        bf2.reshape(1, F), Wo1, bo1.reshape(1, F // 2), Wo2,
        bo2.reshape(1, 1))

    vals = mw.reshape(NW, NCH, CH)
    zeros = jnp.zeros((PT,), dtype=_f32)
    rho2 = _scatter_rho(pidx_w, vals, zeros)
    rho = rho2[0] + rho2[1] + final_bias[0]
    return rho.reshape(B, P_MAX)


# speedup vs baseline: 9.8752x; 1.9055x over previous
import functools
import math

import jax
import jax.numpy as jnp
from jax import lax
from jax.experimental import pallas as pl
from jax.experimental.pallas import tpu as pltpu
from jax.experimental.pallas import tpu_sc as plsc

B = 4
N_MAX = 200
P_MAX = 4000
E_MAX = 50000
F = 64
N_SINC = 20
CUTOFF = 4.0

NPA = 256
ET = 1024
TPB = 49
EPB = TPB * ET
EP = B * EPB
GRID = B * TPB
NW = 32
EPW = EP // NW
CH = 128
NCH = EPW // CH
PT = B * P_MAX
CW = 16

_f32 = jnp.float32
_i32 = jnp.int32


def _sc_mesh():
    return plsc.VectorSubcoreMesh(core_axis_name="c", subcore_axis_name="s")


def _gather_coords(ptab, pidx):
    @functools.partial(
        pl.kernel,
        out_type=jax.ShapeDtypeStruct((EP, CW), _f32),
        mesh=_sc_mesh(),
        scratch_types=[
            pltpu.VMEM((NCH, CH), _i32),
            pltpu.VMEM((CH, CW), _f32),
            pltpu.SemaphoreType.DMA,
        ],
        compiler_params=pltpu.CompilerParams(use_tc_tiling_on_sc=False),
    )
    def k(ptab_h, pidx_h, pe_h, pidx_v, pbuf, psem):
        c = lax.axis_index("c")
        s = lax.axis_index("s")
        wid = s * 2 + c
        pltpu.sync_copy(pidx_h.at[wid], pidx_v)
        base = wid * EPW

        def body(j, carry):
            pltpu.async_copy(ptab_h.at[pidx_v.at[j]], pbuf, psem).wait()
            pltpu.sync_copy(pbuf, pe_h.at[pl.ds(base + j * CH, CH)])
            return carry

        lax.fori_loop(0, NCH, body, 0)

    return k(ptab, pidx)


def _scatter_rho(pidx, vals, zeros):
    @functools.partial(
        pl.kernel,
        out_type=jax.ShapeDtypeStruct((2, PT), _f32),
        mesh=_sc_mesh(),
        scratch_types=[
            pltpu.VMEM((NCH, CH), _i32),
            pltpu.VMEM((NCH, CH), _f32),
            pltpu.VMEM_SHARED((PT,), _f32),
        ],
        compiler_params=pltpu.CompilerParams(use_tc_tiling_on_sc=False),
    )
    def k(pidx_h, vals_h, zeros_h, out_h, pidx_v, vals_v, rho_sh):
        c = lax.axis_index("c")
        s = lax.axis_index("s")
        wid = s * 2 + c

        @pl.when(s == 0)
        def _init():
            pltpu.sync_copy(zeros_h, rho_sh)

        plsc.subcore_barrier()
        pltpu.sync_copy(pidx_h.at[wid], pidx_v)
        pltpu.sync_copy(vals_h.at[wid], vals_v)

        def body(j, carry):
            pltpu.sync_copy(vals_v.at[j], rho_sh.at[pidx_v.at[j]], add=True)
            return carry

        lax.fori_loop(0, NCH, body, 0)
        plsc.subcore_barrier()

        @pl.when(s == 0)
        def _emit():
            pltpu.sync_copy(rho_sh, out_h.at[c])

    return k(pidx, vals, zeros)


def _tc_dense(aidx3, peT, S_pT, V_pT, axyzT, Ws1T, bs1c, ln_gc, ln_bc,
              Ws2T, bs2c, Wf1T, bf1c, Wf2T, bf2c, Wo1T, bo1c, Wo2T, bo2c):
    def body(aidx_ref, peT_ref, S_ref, V_ref, ax_ref, Ws1_ref, bs1_ref,
             lng_ref, lnb_ref, Ws2_ref, bs2_ref, Wf1_ref, bf1_ref, Wf2_ref,
             bf2_ref, Wo1_ref, bo1_ref, Wo2_ref, bo2_ref, mw_ref):
        i = pl.program_id(0)
        t = i % TPB

        V0 = V_ref[0, 0]
        V1 = V_ref[0, 1]
        V2 = V_ref[0, 2]
        S_T = S_ref[0]
        n_T = jnp.sqrt(V0 * V0 + V1 * V1 + V2 * V2 + 1e-8)
        WAT = Ws1_ref[:, 0:64]
        WBT = Ws1_ref[:, 64:128]
        WCT = Ws1_ref[:, 128:192]
        A1T = (jnp.dot(WAT, S_T, preferred_element_type=_f32)
               + jnp.dot(WBT, n_T, preferred_element_type=_f32)
               + bs1_ref[...])
        VW0T = jnp.dot(WCT, V0, preferred_element_type=_f32)
        VW1T = jnp.dot(WCT, V1, preferred_element_type=_f32)
        VW2T = jnp.dot(WCT, V2, preferred_element_type=_f32)
        TT = jnp.concatenate([A1T, VW0T, VW1T, VW2T, ax_ref[0]], axis=0)

        a = aidx_ref[0]
        onehotT = (lax.broadcasted_iota(_i32, (NPA, ET), 0) == a).astype(_f32)
        GT = jnp.dot(TT, onehotT, preferred_element_type=_f32)

        diffT = peT_ref[0] - GT[256:264, :]
        d2 = jnp.sum(diffT * diffT, axis=0, keepdims=True)
        dist = jnp.sqrt(d2)
        inv_safe = 1.0 / jnp.sqrt(d2 + 1e-8)

        u = (GT[0:64, :]
             + (diffT[0:1, :] * inv_safe) * GT[64:128, :]
             + (diffT[1:2, :] * inv_safe) * GT[128:192, :]
             + (diffT[2:3, :] * inv_safe) * GT[192:256, :])
        mu = jnp.mean(u, axis=0, keepdims=True)
        var = jnp.mean((u - mu) ** 2, axis=0, keepdims=True)
        y = (u - mu) / jnp.sqrt(var + 1e-5) * lng_ref[...] + lnb_ref[...]
        sy = y * (1.0 / (1.0 + jnp.exp(-y)))
        state = jnp.dot(Ws2_ref[...], sy, preferred_element_type=_f32) + bs2_ref[...]

        kcol = ((lax.broadcasted_iota(_i32, (N_SINC, 1), 0) + 1).astype(_f32)
                * (math.pi / CUTOFF))
        e_piT = jnp.sin(kcol * dist) * (1.0 / dist)
        f1 = jnp.dot(Wf1_ref[...], e_piT, preferred_element_type=_f32) + bf1_ref[...]
        sf1 = f1 * (1.0 / (1.0 + jnp.exp(-f1)))
        W_piT = jnp.dot(Wf2_ref[...], sf1, preferred_element_type=_f32) + bf2_ref[...]

        m_prime = W_piT * state
        o1 = jnp.dot(Wo1_ref[...], m_prime, preferred_element_type=_f32) + bo1_ref[...]
        so1 = o1 * (1.0 / (1.0 + jnp.exp(-o1)))
        m = jnp.dot(Wo2_ref[...], so1, preferred_element_type=_f32) + bo2_ref[...]

        x = dist * (1.0 / CUTOFF)
        x5 = x * x * x * x * x
        env = 1.0 - 21.0 * x5 + 35.0 * x5 * x - 15.0 * x5 * x * x
        env = jnp.where(dist < CUTOFF, env, 0.0)

        mw = m * env
        eid = t * ET + lax.broadcasted_iota(_i32, (1, ET), 1)
        mw_ref[0] = jnp.where(eid < E_MAX, mw, 0.0)

    grid_spec = pl.GridSpec(
        grid=(GRID,),
        in_specs=[
            pl.BlockSpec((1, 1, ET), lambda i: (i, 0, 0)),
            pl.BlockSpec((1, 8, ET), lambda i: (i, 0, 0)),
            pl.BlockSpec((1, F, NPA), lambda i: (i // TPB, 0, 0)),
            pl.BlockSpec((1, 3, F, NPA), lambda i: (i // TPB, 0, 0, 0)),
            pl.BlockSpec((1, 8, NPA), lambda i: (i // TPB, 0, 0)),
            pl.BlockSpec((F, 192), lambda i: (0, 0)),
            pl.BlockSpec((F, 1), lambda i: (0, 0)),
            pl.BlockSpec((F, 1), lambda i: (0, 0)),
            pl.BlockSpec((F, 1), lambda i: (0, 0)),
            pl.BlockSpec((F, F), lambda i: (0, 0)),
            pl.BlockSpec((F, 1), lambda i: (0, 0)),
            pl.BlockSpec((F, N_SINC), lambda i: (0, 0)),
            pl.BlockSpec((F, 1), lambda i: (0, 0)),
            pl.BlockSpec((F, F), lambda i: (0, 0)),
            pl.BlockSpec((F, 1), lambda i: (0, 0)),
            pl.BlockSpec((F // 2, F), lambda i: (0, 0)),
            pl.BlockSpec((F // 2, 1), lambda i: (0, 0)),
            pl.BlockSpec((1, F // 2), lambda i: (0, 0)),
            pl.BlockSpec((1, 1), lambda i: (0, 0)),
        ],
        out_specs=pl.BlockSpec((1, 1, ET), lambda i: (i, 0, 0)),
    )

    return pl.pallas_call(
        body,
        grid_spec=grid_spec,
        out_shape=jax.ShapeDtypeStruct((GRID, 1, ET), _f32),
    )(aidx3, peT, S_pT, V_pT, axyzT, Ws1T, bs1c, ln_gc, ln_bc, Ws2T, bs2c,
      Wf1T, bf1c, Wf2T, bf2c, Wo1T, bo1c, Wo2T, bo2c)


def kernel(atom_xyz, probe_xyz, cell, probe_edges, probe_edges_displacement,
           num_nodes, num_probes, num_probe_edges, S_JK, V_JK, Ws1, bs1,
           ln_g, ln_b, Ws2, bs2, Wf1, bf1, Wf2, bf2, Wo1, bo1, Wo2, bo2,
           final_bias):
    pad_e = EPB - E_MAX

    a_idx = probe_edges[:, :, 0].astype(_i32)
    p_idx = probe_edges[:, :, 1].astype(_i32)
    a_idx = jnp.pad(a_idx, ((0, 0), (0, pad_e)))
    p_idx = jnp.pad(p_idx, ((0, 0), (0, pad_e)))
    boff = jnp.arange(B, dtype=_i32)[:, None]
    pidx_g = (p_idx + boff * P_MAX).reshape(EP)

    ptab = jnp.pad(probe_xyz.reshape(PT, 3), ((0, 0), (0, CW - 3)))
    pidx_w = pidx_g.reshape(NW, NCH, CH)
    pe = _gather_coords(ptab, pidx_w)
    peT = pe[:, :8].T.reshape(8, GRID, ET).transpose(1, 0, 2)

    S_pT = jnp.pad(S_JK.reshape(B, N_MAX, F).transpose(0, 2, 1),
                   ((0, 0), (0, 0), (0, NPA - N_MAX)))
    V_pT = jnp.pad(V_JK.reshape(B, N_MAX, 3, F).transpose(0, 2, 3, 1),
                   ((0, 0), (0, 0), (0, 0), (0, NPA - N_MAX)))
    axyzT = jnp.pad(atom_xyz.transpose(0, 2, 1),
                    ((0, 0), (0, 8 - 3), (0, NPA - N_MAX)))

    aidx3 = a_idx.reshape(GRID, 1, ET)

    mw = _tc_dense(
        aidx3, peT, S_pT, V_pT, axyzT, Ws1.T,
        bs1.reshape(F, 1), ln_g.reshape(F, 1), ln_b.reshape(F, 1),
        Ws2.T, bs2.reshape(F, 1), Wf1.T, bf1.reshape(F, 1), Wf2.T,
        bf2.reshape(F, 1), Wo1.T, bo1.reshape(F // 2, 1), Wo2.T,
        bo2.reshape(1, 1))

    vals = mw.reshape(NW, NCH, CH)
    zeros = jnp.zeros((PT,), dtype=_f32)
    rho2 = _scatter_rho(pidx_w, vals, zeros)
    rho = rho2[0] + rho2[1] + final_bias[0]
    return rho.reshape(B, P_MAX)


# speedup vs baseline: 12.9059x; 1.3069x over previous
import functools
import math

import jax
import jax.numpy as jnp
from jax import lax
from jax.experimental import pallas as pl
from jax.experimental.pallas import tpu as pltpu
from jax.experimental.pallas import tpu_sc as plsc

B = 4
N_MAX = 200
P_MAX = 4000
E_MAX = 50000
F = 64
N_SINC = 20
CUTOFF = 4.0

NPA = 256
TR = 4 * F + 16
ET = 2048
TPB = 25
EPB = TPB * ET
EP = B * EPB
GRID = B * TPB
NW = 32
EPW = EP // NW
CH = 128
NCH = EPW // CH
PT = B * P_MAX
CW = 16

_f32 = jnp.float32
_bf16 = jnp.bfloat16
_i32 = jnp.int32


def _sc_mesh():
    return plsc.VectorSubcoreMesh(core_axis_name="c", subcore_axis_name="s")


def _gather_coords(ptab, pidx):
    @functools.partial(
        pl.kernel,
        out_type=jax.ShapeDtypeStruct((EP, CW), _f32),
        mesh=_sc_mesh(),
        scratch_types=[
            pltpu.VMEM((NCH, CH), _i32),
            pltpu.VMEM((CH, CW), _f32),
            pltpu.SemaphoreType.DMA,
        ],
        compiler_params=pltpu.CompilerParams(use_tc_tiling_on_sc=False),
    )
    def k(ptab_h, pidx_h, pe_h, pidx_v, pbuf, psem):
        c = lax.axis_index("c")
        s = lax.axis_index("s")
        wid = s * 2 + c
        pltpu.sync_copy(pidx_h.at[wid], pidx_v)
        base = wid * EPW

        def body(j, carry):
            pltpu.async_copy(ptab_h.at[pidx_v.at[j]], pbuf, psem).wait()
            pltpu.sync_copy(pbuf, pe_h.at[pl.ds(base + j * CH, CH)])
            return carry

        lax.fori_loop(0, NCH, body, 0)

    return k(ptab, pidx)


def _scatter_rho(pidx, vals, zeros):
    @functools.partial(
        pl.kernel,
        out_type=jax.ShapeDtypeStruct((2, PT), _f32),
        mesh=_sc_mesh(),
        scratch_types=[
            pltpu.VMEM((NCH, CH), _i32),
            pltpu.VMEM((NCH, CH), _f32),
            pltpu.VMEM_SHARED((PT,), _f32),
        ],
        compiler_params=pltpu.CompilerParams(use_tc_tiling_on_sc=False),
    )
    def k(pidx_h, vals_h, zeros_h, out_h, pidx_v, vals_v, rho_sh):
        c = lax.axis_index("c")
        s = lax.axis_index("s")
        wid = s * 2 + c

        @pl.when(s == 0)
        def _init():
            pltpu.sync_copy(zeros_h, rho_sh)

        plsc.subcore_barrier()
        pltpu.sync_copy(pidx_h.at[wid], pidx_v)
        pltpu.sync_copy(vals_h.at[wid], vals_v)

        def body(j, carry):
            pltpu.sync_copy(vals_v.at[j], rho_sh.at[pidx_v.at[j]], add=True)
            return carry

        lax.fori_loop(0, NCH, body, 0)
        plsc.subcore_barrier()

        @pl.when(s == 0)
        def _emit():
            pltpu.sync_copy(rho_sh, out_h.at[c])

    return k(pidx, vals, zeros)


def _table_prep(S_pT, V_pT, axyzT, Ws1T):
    def body(S_ref, V_ref, ax_ref, Ws1_ref, out_ref):
        V0 = V_ref[0, 0]
        V1 = V_ref[0, 1]
        V2 = V_ref[0, 2]
        S_T = S_ref[0]
        n_T = jnp.sqrt(V0 * V0 + V1 * V1 + V2 * V2 + 1e-8)
        WAT = Ws1_ref[:, 0:64]
        WBT = Ws1_ref[:, 64:128]
        WCT = Ws1_ref[:, 128:192]
        A1T = (jnp.dot(WAT, S_T, preferred_element_type=_f32)
               + jnp.dot(WBT, n_T, preferred_element_type=_f32))
        VW0T = jnp.dot(WCT, V0, preferred_element_type=_f32)
        VW1T = jnp.dot(WCT, V1, preferred_element_type=_f32)
        VW2T = jnp.dot(WCT, V2, preferred_element_type=_f32)
        ax = ax_ref[0]
        hi = ax.astype(_bf16)
        lo = (ax - hi.astype(_f32)).astype(_bf16)
        TT = jnp.concatenate(
            [A1T.astype(_bf16), VW0T.astype(_bf16), VW1T.astype(_bf16),
             VW2T.astype(_bf16), hi, lo], axis=0)
        out_ref[0] = TT

    grid_spec = pl.GridSpec(
        grid=(B,),
        in_specs=[
            pl.BlockSpec((1, F, NPA), lambda b: (b, 0, 0)),
            pl.BlockSpec((1, 3, F, NPA), lambda b: (b, 0, 0, 0)),
            pl.BlockSpec((1, 8, NPA), lambda b: (b, 0, 0)),
            pl.BlockSpec((F, 192), lambda b: (0, 0)),
        ],
        out_specs=pl.BlockSpec((1, TR, NPA), lambda b: (b, 0, 0)),
    )
    return pl.pallas_call(
        body,
        grid_spec=grid_spec,
        out_shape=jax.ShapeDtypeStruct((B, TR, NPA), _bf16),
    )(S_pT, V_pT, axyzT, Ws1T)


def _tc_dense(aidx3, peT, TT_b, iota_bf, bs1c, ln_gc, ln_bc,
              Ws2T, bs2c, Wf1T, bf1c, Wf2T, bf2c, Wo1T, bo1c, Wo2T, bo2c):
    def body(aidx_ref, peT_ref, TT_ref, iota_ref, bs1_ref, lng_ref, lnb_ref,
             Ws2_ref, bs2_ref, Wf1_ref, bf1_ref, Wf2_ref, bf2_ref, Wo1_ref,
             bo1_ref, Wo2_ref, bo2_ref, mw_ref):
        i = pl.program_id(0)
        t = i % TPB

        a = aidx_ref[0].astype(_bf16)
        onehotT = (iota_ref[...] == a).astype(_bf16)
        GT = jnp.dot(TT_ref[0], onehotT, preferred_element_type=_f32)

        axyz = GT[256:264, :] + GT[264:272, :]
        diffT = peT_ref[0] - axyz
        d2 = jnp.sum(diffT * diffT, axis=0, keepdims=True)
        dist = jnp.sqrt(d2)
        inv_safe = 1.0 / jnp.sqrt(d2 + 1e-8)

        u = (GT[0:64, :] + bs1_ref[...]
             + (diffT[0:1, :] * inv_safe) * GT[64:128, :]
             + (diffT[1:2, :] * inv_safe) * GT[128:192, :]
             + (diffT[2:3, :] * inv_safe) * GT[192:256, :])
        mu = jnp.mean(u, axis=0, keepdims=True)
        var = jnp.mean((u - mu) ** 2, axis=0, keepdims=True)
        y = (u - mu) / jnp.sqrt(var + 1e-5) * lng_ref[...] + lnb_ref[...]
        sy = y * (1.0 / (1.0 + jnp.exp(-y)))
        state = jnp.dot(Ws2_ref[...], sy, preferred_element_type=_f32) + bs2_ref[...]

        kcol = ((lax.broadcasted_iota(_i32, (N_SINC, 1), 0) + 1).astype(_f32)
                * (math.pi / CUTOFF))
        ang = kcol * dist
        q = jnp.round(ang * (1.0 / (2.0 * math.pi)))
        rr = (ang - q * 6.2831855) - q * (-1.7484555314695172e-07)
        r2 = rr * rr
        sp = -2.036221212579145e-08
        for cc in (2.6997138291596863e-06, -0.00019808632624911042,
                   0.008332402961152507, -0.16666552631103124,
                   0.9999995999016198):
            sp = sp * r2 + cc
        e_piT = (sp * rr) * (1.0 / dist)
        f1 = jnp.dot(Wf1_ref[...], e_piT, preferred_element_type=_f32) + bf1_ref[...]
        sf1 = f1 * (1.0 / (1.0 + jnp.exp(-f1)))
        W_piT = jnp.dot(Wf2_ref[...], sf1, preferred_element_type=_f32) + bf2_ref[...]

        m_prime = W_piT * state
        o1 = jnp.dot(Wo1_ref[...], m_prime, preferred_element_type=_f32) + bo1_ref[...]
        so1 = o1 * (1.0 / (1.0 + jnp.exp(-o1)))
        m = jnp.dot(Wo2_ref[...], so1, preferred_element_type=_f32) + bo2_ref[...]

        x = dist * (1.0 / CUTOFF)
        x5 = x * x * x * x * x
        env = 1.0 - 21.0 * x5 + 35.0 * x5 * x - 15.0 * x5 * x * x
        env = jnp.where(dist < CUTOFF, env, 0.0)

        mw = m * env
        eid = t * ET + lax.broadcasted_iota(_i32, (1, ET), 1)
        mw_ref[0] = jnp.where(eid < E_MAX, mw, 0.0)

    grid_spec = pl.GridSpec(
        grid=(GRID,),
        in_specs=[
            pl.BlockSpec((1, 1, ET), lambda i: (i, 0, 0)),
            pl.BlockSpec((1, 8, ET), lambda i: (i, 0, 0)),
            pl.BlockSpec((1, TR, NPA), lambda i: (i // TPB, 0, 0)),
            pl.BlockSpec((NPA, ET), lambda i: (0, 0)),
            pl.BlockSpec((F, 1), lambda i: (0, 0)),
            pl.BlockSpec((F, 1), lambda i: (0, 0)),
            pl.BlockSpec((F, 1), lambda i: (0, 0)),
            pl.BlockSpec((F, F), lambda i: (0, 0)),
            pl.BlockSpec((F, 1), lambda i: (0, 0)),
            pl.BlockSpec((F, N_SINC), lambda i: (0, 0)),
            pl.BlockSpec((F, 1), lambda i: (0, 0)),
            pl.BlockSpec((F, F), lambda i: (0, 0)),
            pl.BlockSpec((F, 1), lambda i: (0, 0)),
            pl.BlockSpec((F // 2, F), lambda i: (0, 0)),
            pl.BlockSpec((F // 2, 1), lambda i: (0, 0)),
            pl.BlockSpec((1, F // 2), lambda i: (0, 0)),
            pl.BlockSpec((1, 1), lambda i: (0, 0)),
        ],
        out_specs=pl.BlockSpec((1, 1, ET), lambda i: (i, 0, 0)),
    )

    return pl.pallas_call(
        body,
        grid_spec=grid_spec,
        out_shape=jax.ShapeDtypeStruct((GRID, 1, ET), _f32),
    )(aidx3, peT, TT_b, iota_bf, bs1c, ln_gc, ln_bc, Ws2T, bs2c,
      Wf1T, bf1c, Wf2T, bf2c, Wo1T, bo1c, Wo2T, bo2c)


def kernel(atom_xyz, probe_xyz, cell, probe_edges, probe_edges_displacement,
           num_nodes, num_probes, num_probe_edges, S_JK, V_JK, Ws1, bs1,
           ln_g, ln_b, Ws2, bs2, Wf1, bf1, Wf2, bf2, Wo1, bo1, Wo2, bo2,
           final_bias):
    pad_e = EPB - E_MAX

    a_idx = probe_edges[:, :, 0].astype(_i32)
    p_idx = probe_edges[:, :, 1].astype(_i32)
    a_idx = jnp.pad(a_idx, ((0, 0), (0, pad_e)))
    p_idx = jnp.pad(p_idx, ((0, 0), (0, pad_e)))
    boff = jnp.arange(B, dtype=_i32)[:, None]
    pidx_g = (p_idx + boff * P_MAX).reshape(EP)

    ptab = jnp.pad(probe_xyz.reshape(PT, 3), ((0, 0), (0, CW - 3)))
    pidx_w = pidx_g.reshape(NW, NCH, CH)
    pe = _gather_coords(ptab, pidx_w)
    peT = pe[:, :8].T.reshape(8, GRID, ET).transpose(1, 0, 2)

    S_pT = jnp.pad(S_JK.reshape(B, N_MAX, F).transpose(0, 2, 1),
                   ((0, 0), (0, 0), (0, NPA - N_MAX)))
    V_pT = jnp.pad(V_JK.reshape(B, N_MAX, 3, F).transpose(0, 2, 3, 1),
                   ((0, 0), (0, 0), (0, 0), (0, NPA - N_MAX)))
    axyzT = jnp.pad(atom_xyz.transpose(0, 2, 1),
                    ((0, 0), (0, 8 - 3), (0, NPA - N_MAX)))

    TT_b = _table_prep(S_pT, V_pT, axyzT, Ws1.T)
    iota_bf = jnp.broadcast_to(
        jnp.arange(NPA, dtype=_bf16)[:, None], (NPA, ET))

    aidx3 = a_idx.reshape(GRID, 1, ET)

    mw = _tc_dense(
        aidx3, peT, TT_b, iota_bf,
        bs1.reshape(F, 1), ln_g.reshape(F, 1), ln_b.reshape(F, 1),
        Ws2.T, bs2.reshape(F, 1), Wf1.T, bf1.reshape(F, 1), Wf2.T,
        bf2.reshape(F, 1), Wo1.T, bo1.reshape(F // 2, 1), Wo2.T,
        bo2.reshape(1, 1))

    vals = mw.reshape(NW, NCH, CH)
    zeros = jnp.zeros((PT,), dtype=_f32)
    rho2 = _scatter_rho(pidx_w, vals, zeros)
    rho = rho2[0] + rho2[1] + final_bias[0]
    return rho.reshape(B, P_MAX)


# speedup vs baseline: 22.1298x; 1.7147x over previous
import functools
import math

import jax
import jax.numpy as jnp
from jax import lax
from jax.experimental import pallas as pl
from jax.experimental.pallas import tpu as pltpu
from jax.experimental.pallas import tpu_sc as plsc

B = 4
N_MAX = 200
P_MAX = 4000
E_MAX = 50000
F = 64
N_SINC = 20
CUTOFF = 4.0

NPA = 256
TR = 4 * F + 16
ET = 2048
TPB = 25
EPB = TPB * ET
EP = B * EPB
GRID = B * TPB
NW = 32
EPW = EP // NW
CH = 128
NCH = EPW // CH
PT = B * P_MAX

_f32 = jnp.float32
_bf16 = jnp.bfloat16
_i32 = jnp.int32


def _sc_mesh():
    return plsc.VectorSubcoreMesh(core_axis_name="c", subcore_axis_name="s")


def _gather_coords(pxt, pyt, pzt, pidx):
    @functools.partial(
        pl.kernel,
        out_type=(
            jax.ShapeDtypeStruct((NW, EPW), _f32),
            jax.ShapeDtypeStruct((NW, EPW), _f32),
            jax.ShapeDtypeStruct((NW, EPW), _f32),
        ),
        mesh=_sc_mesh(),
        scratch_types=[
            pltpu.VMEM((EPW,), _i32),
            pltpu.VMEM((EPW,), _f32),
            pltpu.VMEM((EPW,), _f32),
            pltpu.VMEM((EPW,), _f32),
            pltpu.SemaphoreType.DMA,
            pltpu.SemaphoreType.DMA,
            pltpu.SemaphoreType.DMA,
        ],
        compiler_params=pltpu.CompilerParams(use_tc_tiling_on_sc=False),
    )
    def k(pxt_h, pyt_h, pzt_h, pidx_h, ox_h, oy_h, oz_h,
          pidx_v, bx, by, bz, sx, sy_, sz):
        c = lax.axis_index("c")
        s = lax.axis_index("s")
        wid = s * 2 + c
        pltpu.sync_copy(pidx_h.at[wid], pidx_v)
        cx = pltpu.async_copy(pxt_h.at[pidx_v], bx, sx)
        cy = pltpu.async_copy(pyt_h.at[pidx_v], by, sy_)
        cz = pltpu.async_copy(pzt_h.at[pidx_v], bz, sz)
        cx.wait()
        cy.wait()
        cz.wait()
        pltpu.sync_copy(bx, ox_h.at[wid])
        pltpu.sync_copy(by, oy_h.at[wid])
        pltpu.sync_copy(bz, oz_h.at[wid])

    return k(pxt, pyt, pzt, pidx)


def _scatter_rho(pidx, vals, zeros):
    @functools.partial(
        pl.kernel,
        out_type=jax.ShapeDtypeStruct((2, PT), _f32),
        mesh=_sc_mesh(),
        scratch_types=[
            pltpu.VMEM((EPW,), _i32),
            pltpu.VMEM((EPW,), _f32),
            pltpu.VMEM_SHARED((PT,), _f32),
        ],
        compiler_params=pltpu.CompilerParams(use_tc_tiling_on_sc=False),
    )
    def k(pidx_h, vals_h, zeros_h, out_h, pidx_v, vals_v, rho_sh):
        c = lax.axis_index("c")
        s = lax.axis_index("s")
        wid = s * 2 + c

        @pl.when(s == 0)
        def _init():
            pltpu.sync_copy(zeros_h, rho_sh)

        plsc.subcore_barrier()
        pltpu.sync_copy(pidx_h.at[wid], pidx_v)
        pltpu.sync_copy(vals_h.at[wid], vals_v)
        pltpu.sync_copy(vals_v, rho_sh.at[pidx_v], add=True)
        plsc.subcore_barrier()

        @pl.when(s == 0)
        def _emit():
            pltpu.sync_copy(rho_sh, out_h.at[c])

    return k(pidx, vals, zeros)


def _table_prep(S_pT, V_pT, axyzT, Ws1T):
    def body(S_ref, V_ref, ax_ref, Ws1_ref, out_ref):
        V0 = V_ref[0, 0]
        V1 = V_ref[0, 1]
        V2 = V_ref[0, 2]
        S_T = S_ref[0]
        n_T = jnp.sqrt(V0 * V0 + V1 * V1 + V2 * V2 + 1e-8)
        WAT = Ws1_ref[:, 0:64]
        WBT = Ws1_ref[:, 64:128]
        WCT = Ws1_ref[:, 128:192]
        A1T = (jnp.dot(WAT, S_T, preferred_element_type=_f32)
               + jnp.dot(WBT, n_T, preferred_element_type=_f32))
        VW0T = jnp.dot(WCT, V0, preferred_element_type=_f32)
        VW1T = jnp.dot(WCT, V1, preferred_element_type=_f32)
        VW2T = jnp.dot(WCT, V2, preferred_element_type=_f32)
        ax = ax_ref[0]
        hi = ax.astype(_bf16)
        lo = (ax - hi.astype(_f32)).astype(_bf16)
        TT = jnp.concatenate(
            [A1T.astype(_bf16), VW0T.astype(_bf16), VW1T.astype(_bf16),
             VW2T.astype(_bf16), hi, lo], axis=0)
        out_ref[0] = TT

    grid_spec = pl.GridSpec(
        grid=(B,),
        in_specs=[
            pl.BlockSpec((1, F, NPA), lambda b: (b, 0, 0)),
            pl.BlockSpec((1, 3, F, NPA), lambda b: (b, 0, 0, 0)),
            pl.BlockSpec((1, 8, NPA), lambda b: (b, 0, 0)),
            pl.BlockSpec((F, 192), lambda b: (0, 0)),
        ],
        out_specs=pl.BlockSpec((1, TR, NPA), lambda b: (b, 0, 0)),
    )
    return pl.pallas_call(
        body,
        grid_spec=grid_spec,
        out_shape=jax.ShapeDtypeStruct((B, TR, NPA), _bf16),
    )(S_pT, V_pT, axyzT, Ws1T)


def _tc_dense(aidx3, pex3, pey3, pez3, TT_b, iota_bf, bs1c, ln_gc, ln_bc,
              Ws2T, bs2c, Wf1T, bf1c, Wf2T, bf2c, Wo1T, bo1c, Wo2T, bo2c):
    def body(aidx_ref, pex_ref, pey_ref, pez_ref, TT_ref, iota_ref, bs1_ref,
             lng_ref, lnb_ref, Ws2_ref, bs2_ref, Wf1_ref, bf1_ref, Wf2_ref,
             bf2_ref, Wo1_ref, bo1_ref, Wo2_ref, bo2_ref, mw_ref):
        i = pl.program_id(0)
        t = i % TPB

        a = aidx_ref[0].astype(_bf16)
        onehotT = (iota_ref[...] == a).astype(_bf16)
        GT = jnp.dot(TT_ref[0], onehotT, preferred_element_type=_f32)

        axyz = GT[256:264, :] + GT[264:272, :]
        d0 = pex_ref[0] - axyz[0:1, :]
        d1 = pey_ref[0] - axyz[1:2, :]
        d2c = pez_ref[0] - axyz[2:3, :]
        d2 = d0 * d0 + d1 * d1 + d2c * d2c
        dist = jnp.sqrt(d2)
        inv_safe = 1.0 / jnp.sqrt(d2 + 1e-8)

        u = (GT[0:64, :] + bs1_ref[...]
             + (d0 * inv_safe) * GT[64:128, :]
             + (d1 * inv_safe) * GT[128:192, :]
             + (d2c * inv_safe) * GT[192:256, :])
        mu = jnp.mean(u, axis=0, keepdims=True)
        var = jnp.mean((u - mu) ** 2, axis=0, keepdims=True)
        y = (u - mu) / jnp.sqrt(var + 1e-5) * lng_ref[...] + lnb_ref[...]
        sy = y * (1.0 / (1.0 + jnp.exp(-y)))
        state = jnp.dot(Ws2_ref[...], sy, preferred_element_type=_f32) + bs2_ref[...]

        kcol = ((lax.broadcasted_iota(_i32, (N_SINC, 1), 0) + 1).astype(_f32)
                * (math.pi / CUTOFF))
        ang = kcol * dist
        q = jnp.round(ang * (1.0 / (2.0 * math.pi)))
        rr = (ang - q * 6.2831855) - q * (-1.7484555314695172e-07)
        r2 = rr * rr
        sp = -2.036221212579145e-08
        for cc in (2.6997138291596863e-06, -0.00019808632624911042,
                   0.008332402961152507, -0.16666552631103124,
                   0.9999995999016198):
            sp = sp * r2 + cc
        e_piT = (sp * rr) * (1.0 / dist)
        f1 = jnp.dot(Wf1_ref[...], e_piT, preferred_element_type=_f32) + bf1_ref[...]
        sf1 = f1 * (1.0 / (1.0 + jnp.exp(-f1)))
        W_piT = jnp.dot(Wf2_ref[...], sf1, preferred_element_type=_f32) + bf2_ref[...]

        m_prime = W_piT * state
        o1 = jnp.dot(Wo1_ref[...], m_prime, preferred_element_type=_f32) + bo1_ref[...]
        so1 = o1 * (1.0 / (1.0 + jnp.exp(-o1)))
        m = jnp.dot(Wo2_ref[...], so1, preferred_element_type=_f32) + bo2_ref[...]

        x = dist * (1.0 / CUTOFF)
        x5 = x * x * x * x * x
        env = 1.0 - 21.0 * x5 + 35.0 * x5 * x - 15.0 * x5 * x * x
        env = jnp.where(dist < CUTOFF, env, 0.0)

        mw = m * env
        eid = t * ET + lax.broadcasted_iota(_i32, (1, ET), 1)
        mw_ref[0] = jnp.where(eid < E_MAX, mw, 0.0)

    grid_spec = pl.GridSpec(
        grid=(GRID,),
        in_specs=[
            pl.BlockSpec((1, 1, ET), lambda i: (i, 0, 0)),
            pl.BlockSpec((1, 1, ET), lambda i: (i, 0, 0)),
            pl.BlockSpec((1, 1, ET), lambda i: (i, 0, 0)),
            pl.BlockSpec((1, 1, ET), lambda i: (i, 0, 0)),
            pl.BlockSpec((1, TR, NPA), lambda i: (i // TPB, 0, 0)),
            pl.BlockSpec((NPA, ET), lambda i: (0, 0)),
            pl.BlockSpec((F, 1), lambda i: (0, 0)),
            pl.BlockSpec((F, 1), lambda i: (0, 0)),
            pl.BlockSpec((F, 1), lambda i: (0, 0)),
            pl.BlockSpec((F, F), lambda i: (0, 0)),
            pl.BlockSpec((F, 1), lambda i: (0, 0)),
            pl.BlockSpec((F, N_SINC), lambda i: (0, 0)),
            pl.BlockSpec((F, 1), lambda i: (0, 0)),
            pl.BlockSpec((F, F), lambda i: (0, 0)),
            pl.BlockSpec((F, 1), lambda i: (0, 0)),
            pl.BlockSpec((F // 2, F), lambda i: (0, 0)),
            pl.BlockSpec((F // 2, 1), lambda i: (0, 0)),
            pl.BlockSpec((1, F // 2), lambda i: (0, 0)),
            pl.BlockSpec((1, 1), lambda i: (0, 0)),
        ],
        out_specs=pl.BlockSpec((1, 1, ET), lambda i: (i, 0, 0)),
    )

    return pl.pallas_call(
        body,
        grid_spec=grid_spec,
        out_shape=jax.ShapeDtypeStruct((GRID, 1, ET), _f32),
    )(aidx3, pex3, pey3, pez3, TT_b, iota_bf, bs1c, ln_gc, ln_bc, Ws2T, bs2c,
      Wf1T, bf1c, Wf2T, bf2c, Wo1T, bo1c, Wo2T, bo2c)


def kernel(atom_xyz, probe_xyz, cell, probe_edges, probe_edges_displacement,
           num_nodes, num_probes, num_probe_edges, S_JK, V_JK, Ws1, bs1,
           ln_g, ln_b, Ws2, bs2, Wf1, bf1, Wf2, bf2, Wo1, bo1, Wo2, bo2,
           final_bias):
    pad_e = EPB - E_MAX

    a_idx = probe_edges[:, :, 0].astype(_i32)
    p_idx = probe_edges[:, :, 1].astype(_i32)
    a_idx = jnp.pad(a_idx, ((0, 0), (0, pad_e)))
    p_idx = jnp.pad(p_idx, ((0, 0), (0, pad_e)))
    boff = jnp.arange(B, dtype=_i32)[:, None]
    pidx_g = (p_idx + boff * P_MAX).reshape(EP)

    pxyz_flat = probe_xyz.reshape(PT, 3)
    pxt = pxyz_flat[:, 0]
    pyt = pxyz_flat[:, 1]
    pzt = pxyz_flat[:, 2]
    pidx_w = pidx_g.reshape(NW, EPW)
    pex, pey, pez = _gather_coords(pxt, pyt, pzt, pidx_w)

    S_pT = jnp.pad(S_JK.reshape(B, N_MAX, F).transpose(0, 2, 1),
                   ((0, 0), (0, 0), (0, NPA - N_MAX)))
    V_pT = jnp.pad(V_JK.reshape(B, N_MAX, 3, F).transpose(0, 2, 3, 1),
                   ((0, 0), (0, 0), (0, 0), (0, NPA - N_MAX)))
    axyzT = jnp.pad(atom_xyz.transpose(0, 2, 1),
                    ((0, 0), (0, 8 - 3), (0, NPA - N_MAX)))

    TT_b = _table_prep(S_pT, V_pT, axyzT, Ws1.T)
    iota_bf = jnp.broadcast_to(
        jnp.arange(NPA, dtype=_bf16)[:, None], (NPA, ET))

    aidx3 = a_idx.reshape(GRID, 1, ET)

    mw = _tc_dense(
        aidx3, pex.reshape(GRID, 1, ET), pey.reshape(GRID, 1, ET),
        pez.reshape(GRID, 1, ET), TT_b, iota_bf,
        bs1.reshape(F, 1), ln_g.reshape(F, 1), ln_b.reshape(F, 1),
        Ws2.T, bs2.reshape(F, 1), Wf1.T, bf1.reshape(F, 1), Wf2.T,
        bf2.reshape(F, 1), Wo1.T, bo1.reshape(F // 2, 1), Wo2.T,
        bo2.reshape(1, 1))

    vals = mw.reshape(NW, EPW)
    zeros = jnp.zeros((PT,), dtype=_f32)
    rho2 = _scatter_rho(pidx_w, vals, zeros)
    rho = rho2[0] + rho2[1] + final_bias[0]
    return rho.reshape(B, P_MAX)
